# baseline (speedup 1.0000x reference)
import jax
import jax.numpy as jnp
from jax.experimental import pallas as pl
from jax.experimental.pallas import tpu as pltpu

_R = 128
_C = 128
_B = _R * _C


def _partner(x, j, bit_is_zero):
    if j < _C:
        down = pltpu.roll(x, _C - j, axis=1)
        up = pltpu.roll(x, j, axis=1)
    else:
        sj = j // _C
        down = pltpu.roll(x, _R - sj, axis=0)
        up = pltpu.roll(x, sj, axis=0)
    return jnp.where(bit_is_zero, down, up)


def _cox_body(lh_ref, et_ref, ei_ref, out_ref):
    key = et_ref[...]
    lh = lh_ref[...]
    ei_f = ei_ref[...]

    ri = jax.lax.broadcasted_iota(jnp.int32, (_R, _C), 0)
    ci = jax.lax.broadcasted_iota(jnp.int32, (_R, _C), 1)
    pos = ri * _C + ci

    n_ev = jnp.sum(ei_f)
    sum_ei_lh = jnp.sum(ei_f * lh)
    m = jnp.max(lh)

    lh_bits = jax.lax.bitcast_convert_type(lh, jnp.int32)
    lh16 = ((lh_bits + 0x8000) >> 16) & 0xFFFF
    pack = (pos << 17) | (ei_f.astype(jnp.int32) << 16) | lh16

    def exchange(key, pack, pk, pp, swap_mask):
        p_self = (key > pk) | ((key == pk) & (pack < pp))
        take_partner = p_self == swap_mask
        return (jnp.where(take_partner, pk, key),
                jnp.where(take_partner, pp, pack))

    def lane_stages(key, pack, asc, j_hi):
        def body(t, carry):
            key, pack = carry
            j = jnp.int32(j_hi) >> t
            bit0 = (ci & j) == 0
            swap_mask = bit0 != asc
            pk = jnp.where(bit0, pltpu.roll(key, _C - j, axis=1),
                           pltpu.roll(key, j, axis=1))
            pp = jnp.where(bit0, pltpu.roll(pack, _C - j, axis=1),
                           pltpu.roll(pack, j, axis=1))
            return exchange(key, pack, pk, pp, swap_mask)
        n = j_hi.bit_length()
        return jax.lax.fori_loop(0, n, body, (key, pack))

    k = 2
    while k <= _B:
        j = k // 2
        if k >= _B:
            asc = jnp.full((_R, _C), True)
        elif k < _C:
            asc = (ci & k) == 0
        else:
            asc = (ri & (k // _C)) == 0
        while j >= _C:
            sj = j // _C
            bit0 = (ri & sj) == 0
            swap_mask = bit0 != asc
            pk = jnp.where(bit0, pltpu.roll(key, _R - sj, axis=0),
                           pltpu.roll(key, sj, axis=0))
            pp = jnp.where(bit0, pltpu.roll(pack, _R - sj, axis=0),
                           pltpu.roll(pack, sj, axis=0))
            key, pack = exchange(key, pack, pk, pp, swap_mask)
            j //= 2
        key, pack = lane_stages(key, pack, asc, j)
        k *= 2

    ei = ((pack >> 16) & 1).astype(jnp.float32)
    lh_s = jax.lax.bitcast_convert_type((pack & 0xFFFF) << 16, jnp.float32)

    e = jnp.exp(lh_s - m)
    cs = e
    t = 1
    while t < _C:
        cs = cs + jnp.where(ci >= t, pltpu.roll(cs, t, axis=1), 0.0)
        t *= 2
    row_tot = pltpu.roll(cs, 1, axis=1)
    row_tot = jnp.where(ci == 0, row_tot, 0.0)
    row_tot = jnp.broadcast_to(jnp.max(row_tot, axis=1, keepdims=True), (_R, _C))
    rp = row_tot
    t = 1
    while t < _R:
        rp = rp + jnp.where(ri >= t, pltpu.roll(rp, t, axis=0), 0.0)
        t *= 2
    prev_rows = rp - row_tot

    sum_ei_logcs = jnp.sum(ei * jnp.log(cs + prev_rows))
    wll = sum_ei_lh - n_ev * m - sum_ei_logcs
    loss = jnp.where(n_ev == 0.0, 0.0, -wll / n_ev)
    out_ref[0, 0] = loss


def kernel(log_hazard, event_times, event_indicators):
    lh = log_hazard.reshape(_R, _C)
    et = event_times.reshape(_R, _C)
    ei = event_indicators.reshape(_R, _C)
    out = pl.pallas_call(
        _cox_body,
        out_shape=jax.ShapeDtypeStruct((1, 1), jnp.float32),
        in_specs=[
            pl.BlockSpec(memory_space=pltpu.VMEM),
            pl.BlockSpec(memory_space=pltpu.VMEM),
            pl.BlockSpec(memory_space=pltpu.VMEM),
        ],
        out_specs=pl.BlockSpec(memory_space=pltpu.SMEM),
    )(lh, et, ei)
    return out[0, 0]


# speedup vs baseline: 1.5066x; 1.5066x over previous
import jax
import jax.numpy as jnp
from jax.experimental import pallas as pl
from jax.experimental.pallas import tpu as pltpu

_R = 128
_C = 128
_B = _R * _C


def _partner(x, j, bit_is_zero):
    if j < _C:
        down = pltpu.roll(x, _C - j, axis=1)
        up = pltpu.roll(x, j, axis=1)
    else:
        sj = j // _C
        down = pltpu.roll(x, _R - sj, axis=0)
        up = pltpu.roll(x, sj, axis=0)
    return jnp.where(bit_is_zero, down, up)


def _cox_body(lh_ref, et_ref, ei_ref, out_ref):
    key = et_ref[...]
    lh = lh_ref[...]
    ei_f = ei_ref[...]

    ri = jax.lax.broadcasted_iota(jnp.int32, (_R, _C), 0)
    ci = jax.lax.broadcasted_iota(jnp.int32, (_R, _C), 1)
    pos = ci * _R + ri
    orig = ri * _C + ci

    n_ev = jnp.sum(ei_f)
    sum_ei_lh = jnp.sum(ei_f * lh)
    m = jnp.max(lh)

    lh_bits = jax.lax.bitcast_convert_type(lh, jnp.int32)
    lh16 = ((lh_bits + 0x8000) >> 16) & 0xFFFF
    pack = (orig << 17) | (ei_f.astype(jnp.int32) << 16) | lh16

    k = 2
    while k <= _B:
        j = k // 2
        while j >= 1:
            bit0 = (pos & j) == 0
            if k >= _B:
                swap_mask = ~bit0
            else:
                swap_mask = bit0 != ((pos & k) == 0)
            if j < _R:
                pk = jnp.where(bit0, pltpu.roll(key, _R - j, axis=0),
                               pltpu.roll(key, j, axis=0))
                pp = jnp.where(bit0, pltpu.roll(pack, _R - j, axis=0),
                               pltpu.roll(pack, j, axis=0))
            else:
                sj = j // _R
                pk = jnp.where(bit0, pltpu.roll(key, _C - sj, axis=1),
                               pltpu.roll(key, sj, axis=1))
                pp = jnp.where(bit0, pltpu.roll(pack, _C - sj, axis=1),
                               pltpu.roll(pack, sj, axis=1))
            p_self = (key > pk) | ((key == pk) & (pack < pp))
            take_partner = p_self == swap_mask
            key = jnp.where(take_partner, pk, key)
            pack = jnp.where(take_partner, pp, pack)
            j //= 2
        k *= 2

    ei = ((pack >> 16) & 1).astype(jnp.float32)
    lh_s = jax.lax.bitcast_convert_type((pack & 0xFFFF) << 16, jnp.float32)

    e = jnp.exp(lh_s - m)
    cs = e
    t = 1
    while t < _R:
        cs = cs + jnp.where(ri >= t, pltpu.roll(cs, t, axis=0), 0.0)
        t *= 2
    col_tot = pltpu.roll(cs, 1, axis=0)
    col_tot = jnp.where(ri == 0, col_tot, 0.0)
    col_tot = jnp.broadcast_to(jnp.max(col_tot, axis=0, keepdims=True), (_R, _C))
    rp = col_tot
    t = 1
    while t < _C:
        rp = rp + jnp.where(ci >= t, pltpu.roll(rp, t, axis=1), 0.0)
        t *= 2
    prev_rows = rp - col_tot

    sum_ei_logcs = jnp.sum(ei * jnp.log(cs + prev_rows))
    wll = sum_ei_lh - n_ev * m - sum_ei_logcs
    loss = jnp.where(n_ev == 0.0, 0.0, -wll / n_ev)
    out_ref[0, 0] = loss


def kernel(log_hazard, event_times, event_indicators):
    lh = log_hazard.reshape(_R, _C)
    et = event_times.reshape(_R, _C)
    ei = event_indicators.reshape(_R, _C)
    out = pl.pallas_call(
        _cox_body,
        out_shape=jax.ShapeDtypeStruct((1, 1), jnp.float32),
        in_specs=[
            pl.BlockSpec(memory_space=pltpu.VMEM),
            pl.BlockSpec(memory_space=pltpu.VMEM),
            pl.BlockSpec(memory_space=pltpu.VMEM),
        ],
        out_specs=pl.BlockSpec(memory_space=pltpu.SMEM),
    )(lh, et, ei)
    return out[0, 0]


# speedup vs baseline: 2.0854x; 1.3842x over previous
import jax
import jax.numpy as jnp
from jax.experimental import pallas as pl
from jax.experimental.pallas import tpu as pltpu

_R = 128
_C = 128
_B = _R * _C
_G = 8
_S = _R // _G


def _cox_body(lh_ref, et_ref, ei_ref, out_ref):
    lh_full = lh_ref[...]
    ei_full = ei_ref[...]

    n_ev = jnp.sum(ei_full)
    sum_ei_lh = jnp.sum(ei_full * lh_full)
    m = jnp.max(lh_full)

    rl = jax.lax.broadcasted_iota(jnp.int32, (_S, _C), 0)
    ci = jax.lax.broadcasted_iota(jnp.int32, (_S, _C), 1)

    keys = []
    packs = []
    for g in range(_G):
        lh_g = lh_ref[g * _S:(g + 1) * _S, :]
        ei_g = ei_ref[g * _S:(g + 1) * _S, :].astype(jnp.int32)
        keys.append(et_ref[g * _S:(g + 1) * _S, :])
        orig = (g * _S + rl) * _C + ci
        lh_bits = jax.lax.bitcast_convert_type(lh_g, jnp.int32)
        lh16 = ((lh_bits + 0x8000) >> 16) & 0xFFFF
        packs.append((orig << 17) | (ei_g << 16) | lh16)

    def exchange(key, pack, pk, pp, swap_mask):
        p_self = (key > pk) | ((key == pk) & (pack < pp))
        if swap_mask is True:
            take = p_self
        elif swap_mask is False:
            take = ~p_self
        else:
            take = p_self == swap_mask
        return jnp.where(take, pk, key), jnp.where(take, pp, pack)

    k = 2
    while k <= _B:
        j = k // 2
        while j >= 1:
            if k >= _B:
                asc_kind, asc_val = "const", True
            elif k >= _C:
                asc_kind, asc_val = "lane", (ci & (k // _C)) == 0
            elif k >= _S:
                asc_kind, asc_val = "group", k // _S
            else:
                asc_kind, asc_val = "sub", (rl & k) == 0

            if j >= _C:
                sj = j // _C
                bit0 = (ci & sj) == 0
                for g in range(_G):
                    if asc_kind == "const":
                        sw = ~bit0
                    elif asc_kind == "lane":
                        sw = bit0 != asc_val
                    elif asc_kind == "group":
                        sw = bit0 if (g & asc_val) else ~bit0
                    else:
                        sw = bit0 != asc_val
                    pk = jnp.where(bit0, pltpu.roll(keys[g], _C - sj, axis=1),
                                   pltpu.roll(keys[g], sj, axis=1))
                    pp = jnp.where(bit0, pltpu.roll(packs[g], _C - sj, axis=1),
                                   pltpu.roll(packs[g], sj, axis=1))
                    keys[g], packs[g] = exchange(keys[g], packs[g], pk, pp, sw)
            elif j >= _S:
                gj = j // _S
                new_keys = list(keys)
                new_packs = list(packs)
                for g in range(_G):
                    h = g ^ gj
                    bit0_const = (g & gj) == 0
                    if asc_kind == "lane":
                        sw = (~asc_val) if bit0_const else asc_val
                    else:
                        asc_const = (True if asc_kind == "const"
                                     else (g & asc_val) == 0)
                        sw = bool(bit0_const != asc_const)
                    new_keys[g], new_packs[g] = exchange(
                        keys[g], packs[g], keys[h], packs[h], sw)
                keys, packs = new_keys, new_packs
            else:
                bit0 = (rl & j) == 0
                for g in range(_G):
                    if asc_kind == "const":
                        sw = ~bit0
                    elif asc_kind == "group":
                        sw = bit0 if (g & asc_val) else ~bit0
                    elif asc_kind == "lane":
                        sw = bit0 != asc_val
                    else:
                        sw = bit0 != asc_val
                    pk = jnp.where(bit0, pltpu.roll(keys[g], _S - j, axis=0),
                                   pltpu.roll(keys[g], j, axis=0))
                    pp = jnp.where(bit0, pltpu.roll(packs[g], _S - j, axis=0),
                                   pltpu.roll(packs[g], j, axis=0))
                    keys[g], packs[g] = exchange(keys[g], packs[g], pk, pp, sw)
            j //= 2
        k *= 2

    running = jnp.zeros((1, _C), jnp.float32)
    css = []
    eis = []
    for g in range(_G):
        ei_g = ((packs[g] >> 16) & 1).astype(jnp.float32)
        lh_g = jax.lax.bitcast_convert_type((packs[g] & 0xFFFF) << 16,
                                            jnp.float32)
        e = jnp.exp(lh_g - m)
        cs = e
        t = 1
        while t < _S:
            cs = cs + jnp.where(rl >= t, pltpu.roll(cs, t, axis=0), 0.0)
            t *= 2
        cs = cs + running
        css.append(cs)
        eis.append(ei_g)
        running = jnp.max(jnp.where(rl == _S - 1, cs, 0.0), axis=0,
                          keepdims=True)

    ct = jnp.broadcast_to(running, (_S, _C))
    t = 1
    while t < _C:
        ct = ct + jnp.where(ci >= t, pltpu.roll(ct, t, axis=1), 0.0)
        t *= 2
    prev_cols = ct - running

    sum_ei_logcs = jnp.float32(0.0)
    for g in range(_G):
        sum_ei_logcs = sum_ei_logcs + jnp.sum(
            eis[g] * jnp.log(css[g] + prev_cols))

    wll = sum_ei_lh - n_ev * m - sum_ei_logcs
    loss = jnp.where(n_ev == 0.0, 0.0, -wll / n_ev)
    out_ref[0, 0] = loss


def kernel(log_hazard, event_times, event_indicators):
    lh = log_hazard.reshape(_R, _C)
    et = event_times.reshape(_R, _C)
    ei = event_indicators.reshape(_R, _C)
    out = pl.pallas_call(
        _cox_body,
        out_shape=jax.ShapeDtypeStruct((1, 1), jnp.float32),
        in_specs=[
            pl.BlockSpec(memory_space=pltpu.VMEM),
            pl.BlockSpec(memory_space=pltpu.VMEM),
            pl.BlockSpec(memory_space=pltpu.VMEM),
        ],
        out_specs=pl.BlockSpec(memory_space=pltpu.SMEM),
    )(lh, et, ei)
    return out[0, 0]


# speedup vs baseline: 2.3745x; 1.1387x over previous
import jax
import jax.numpy as jnp
from jax.experimental import pallas as pl
from jax.experimental.pallas import tpu as pltpu

_R = 128
_C = 128
_B = _R * _C
_G = 16
_S = _R // _G


def _cox_body(lh_ref, et_ref, ei_ref, out_ref):
    lh_full = lh_ref[...]
    ei_full = ei_ref[...]

    n_ev = jnp.sum(ei_full)
    sum_ei_lh = jnp.sum(ei_full * lh_full)
    m = jnp.max(lh_full)

    rl = jax.lax.broadcasted_iota(jnp.int32, (_S, _C), 0)
    ci = jax.lax.broadcasted_iota(jnp.int32, (_S, _C), 1)

    keys = []
    packs = []
    for g in range(_G):
        lh_g = lh_ref[g * _S:(g + 1) * _S, :]
        ei_g = ei_ref[g * _S:(g + 1) * _S, :].astype(jnp.int32)
        keys.append(et_ref[g * _S:(g + 1) * _S, :])
        orig = (g * _S + rl) * _C + ci
        lh_bits = jax.lax.bitcast_convert_type(lh_g, jnp.int32)
        lh16 = ((lh_bits + 0x8000) >> 16) & 0xFFFF
        packs.append((orig << 17) | (ei_g << 16) | lh16)

    def exchange(key, pack, pk, pp, swap_mask):
        p_self = (key > pk) | ((key == pk) & (pack < pp))
        if swap_mask is True:
            take = p_self
        elif swap_mask is False:
            take = ~p_self
        else:
            take = p_self == swap_mask
        return jnp.where(take, pk, key), jnp.where(take, pp, pack)

    k = 2
    while k <= _B:
        j = k // 2
        while j >= 1:
            if k >= _B:
                asc_kind, asc_val = "const", True
            elif k >= _C:
                asc_kind, asc_val = "lane", (ci & (k // _C)) == 0
            elif k >= _S:
                asc_kind, asc_val = "group", k // _S
            else:
                asc_kind, asc_val = "sub", (rl & k) == 0

            if j >= _C:
                sj = j // _C
                bit0 = (ci & sj) == 0
                for g in range(_G):
                    if asc_kind == "const":
                        sw = ~bit0
                    elif asc_kind == "lane":
                        sw = bit0 != asc_val
                    elif asc_kind == "group":
                        sw = bit0 if (g & asc_val) else ~bit0
                    else:
                        sw = bit0 != asc_val
                    pk = jnp.where(bit0, pltpu.roll(keys[g], _C - sj, axis=1),
                                   pltpu.roll(keys[g], sj, axis=1))
                    pp = jnp.where(bit0, pltpu.roll(packs[g], _C - sj, axis=1),
                                   pltpu.roll(packs[g], sj, axis=1))
                    keys[g], packs[g] = exchange(keys[g], packs[g], pk, pp, sw)
            elif j >= _S:
                gj = j // _S
                new_keys = list(keys)
                new_packs = list(packs)
                for g in range(_G):
                    h = g ^ gj
                    bit0_const = (g & gj) == 0
                    if asc_kind == "lane":
                        sw = (~asc_val) if bit0_const else asc_val
                    else:
                        asc_const = (True if asc_kind == "const"
                                     else (g & asc_val) == 0)
                        sw = bool(bit0_const != asc_const)
                    new_keys[g], new_packs[g] = exchange(
                        keys[g], packs[g], keys[h], packs[h], sw)
                keys, packs = new_keys, new_packs
            else:
                bit0 = (rl & j) == 0
                for g in range(_G):
                    if asc_kind == "const":
                        sw = ~bit0
                    elif asc_kind == "group":
                        sw = bit0 if (g & asc_val) else ~bit0
                    elif asc_kind == "lane":
                        sw = bit0 != asc_val
                    else:
                        sw = bit0 != asc_val
                    pk = jnp.where(bit0, pltpu.roll(keys[g], _S - j, axis=0),
                                   pltpu.roll(keys[g], j, axis=0))
                    pp = jnp.where(bit0, pltpu.roll(packs[g], _S - j, axis=0),
                                   pltpu.roll(packs[g], j, axis=0))
                    keys[g], packs[g] = exchange(keys[g], packs[g], pk, pp, sw)
            j //= 2
        k *= 2

    running = jnp.zeros((1, _C), jnp.float32)
    css = []
    eis = []
    for g in range(_G):
        ei_g = ((packs[g] >> 16) & 1).astype(jnp.float32)
        lh_g = jax.lax.bitcast_convert_type((packs[g] & 0xFFFF) << 16,
                                            jnp.float32)
        e = jnp.exp(lh_g - m)
        cs = e
        t = 1
        while t < _S:
            cs = cs + jnp.where(rl >= t, pltpu.roll(cs, t, axis=0), 0.0)
            t *= 2
        cs = cs + running
        css.append(cs)
        eis.append(ei_g)
        running = jnp.max(jnp.where(rl == _S - 1, cs, 0.0), axis=0,
                          keepdims=True)

    ct = jnp.broadcast_to(running, (_S, _C))
    t = 1
    while t < _C:
        ct = ct + jnp.where(ci >= t, pltpu.roll(ct, t, axis=1), 0.0)
        t *= 2
    prev_cols = ct - running

    sum_ei_logcs = jnp.float32(0.0)
    for g in range(_G):
        sum_ei_logcs = sum_ei_logcs + jnp.sum(
            eis[g] * jnp.log(css[g] + prev_cols))

    wll = sum_ei_lh - n_ev * m - sum_ei_logcs
    loss = jnp.where(n_ev == 0.0, 0.0, -wll / n_ev)
    out_ref[0, 0] = loss


def kernel(log_hazard, event_times, event_indicators):
    lh = log_hazard.reshape(_R, _C)
    et = event_times.reshape(_R, _C)
    ei = event_indicators.reshape(_R, _C)
    out = pl.pallas_call(
        _cox_body,
        out_shape=jax.ShapeDtypeStruct((1, 1), jnp.float32),
        in_specs=[
            pl.BlockSpec(memory_space=pltpu.VMEM),
            pl.BlockSpec(memory_space=pltpu.VMEM),
            pl.BlockSpec(memory_space=pltpu.VMEM),
        ],
        out_specs=pl.BlockSpec(memory_space=pltpu.SMEM),
    )(lh, et, ei)
    return out[0, 0]


# speedup vs baseline: 2.3817x; 1.0030x over previous
import jax
import jax.numpy as jnp
from jax.experimental import pallas as pl
from jax.experimental.pallas import tpu as pltpu

_R = 128
_C = 128
_B = _R * _C
_G = 16
_S = _R // _G


def _cox_body(lh_ref, et_ref, ei_ref, out_ref):
    lh_full = lh_ref[...]
    ei_full = ei_ref[...]

    n_ev = jnp.sum(ei_full)
    sum_ei_lh = jnp.sum(ei_full * lh_full)
    m = jnp.max(lh_full)

    rl = jax.lax.broadcasted_iota(jnp.int32, (_S, _C), 0)
    ci = jax.lax.broadcasted_iota(jnp.int32, (_S, _C), 1)

    keys = []
    packs = []
    for g in range(_G):
        lh_g = lh_ref[g * _S:(g + 1) * _S, :]
        ei_g = ei_ref[g * _S:(g + 1) * _S, :].astype(jnp.int32)
        keys.append(et_ref[g * _S:(g + 1) * _S, :])
        orig = (g * _S + rl) * _C + ci
        lh_bits = jax.lax.bitcast_convert_type(lh_g, jnp.int32)
        lh16 = ((lh_bits + 0x8000) >> 16) & 0xFFFF
        packs.append((orig << 17) | (ei_g << 16) | lh16)

    def exchange(key, pack, pk, pp, swap_mask):
        p_self = (key > pk) | ((key == pk) & (pack < pp))
        if swap_mask is True:
            return jnp.where(p_self, pk, key), jnp.where(p_self, pp, pack)
        if swap_mask is False:
            return jnp.where(p_self, key, pk), jnp.where(p_self, pack, pp)
        take = p_self == swap_mask
        return jnp.where(take, pk, key), jnp.where(take, pp, pack)

    k = 2
    while k <= _B:
        j = k // 2
        while j >= 1:
            if k >= _B:
                asc_kind, asc_val = "const", True
            elif k >= _C:
                asc_kind, asc_val = "lane", (ci & (k // _C)) == 0
            elif k >= _S:
                asc_kind, asc_val = "group", k // _S
            else:
                asc_kind, asc_val = "sub", (rl & k) == 0

            if j >= _C:
                sj = j // _C
                bit0 = (ci & sj) == 0
                for g in range(_G):
                    if asc_kind == "const":
                        sw = ~bit0
                    elif asc_kind == "lane":
                        sw = bit0 != asc_val
                    elif asc_kind == "group":
                        sw = bit0 if (g & asc_val) else ~bit0
                    else:
                        sw = bit0 != asc_val
                    pk = jnp.where(bit0, pltpu.roll(keys[g], _C - sj, axis=1),
                                   pltpu.roll(keys[g], sj, axis=1))
                    pp = jnp.where(bit0, pltpu.roll(packs[g], _C - sj, axis=1),
                                   pltpu.roll(packs[g], sj, axis=1))
                    keys[g], packs[g] = exchange(keys[g], packs[g], pk, pp, sw)
            elif j >= _S:
                gj = j // _S
                new_keys = list(keys)
                new_packs = list(packs)
                for g in range(_G):
                    h = g ^ gj
                    bit0_const = (g & gj) == 0
                    if asc_kind == "lane":
                        sw = (~asc_val) if bit0_const else asc_val
                    else:
                        asc_const = (True if asc_kind == "const"
                                     else (g & asc_val) == 0)
                        sw = bool(bit0_const != asc_const)
                    new_keys[g], new_packs[g] = exchange(
                        keys[g], packs[g], keys[h], packs[h], sw)
                keys, packs = new_keys, new_packs
            else:
                bit0 = (rl & j) == 0
                for g in range(_G):
                    if asc_kind == "const":
                        sw = ~bit0
                    elif asc_kind == "group":
                        sw = bit0 if (g & asc_val) else ~bit0
                    elif asc_kind == "lane":
                        sw = bit0 != asc_val
                    else:
                        sw = bit0 != asc_val
                    pk = jnp.where(bit0, pltpu.roll(keys[g], _S - j, axis=0),
                                   pltpu.roll(keys[g], j, axis=0))
                    pp = jnp.where(bit0, pltpu.roll(packs[g], _S - j, axis=0),
                                   pltpu.roll(packs[g], j, axis=0))
                    keys[g], packs[g] = exchange(keys[g], packs[g], pk, pp, sw)
            j //= 2
        k *= 2

    running = jnp.zeros((1, _C), jnp.float32)
    css = []
    eis = []
    for g in range(_G):
        ei_g = ((packs[g] >> 16) & 1).astype(jnp.float32)
        lh_g = jax.lax.bitcast_convert_type((packs[g] & 0xFFFF) << 16,
                                            jnp.float32)
        e = jnp.exp(lh_g - m)
        cs = e
        t = 1
        while t < _S:
            cs = cs + jnp.where(rl >= t, pltpu.roll(cs, t, axis=0), 0.0)
            t *= 2
        cs = cs + running
        css.append(cs)
        eis.append(ei_g)
        running = jnp.max(jnp.where(rl == _S - 1, cs, 0.0), axis=0,
                          keepdims=True)

    ct = jnp.broadcast_to(running, (_S, _C))
    t = 1
    while t < _C:
        ct = ct + jnp.where(ci >= t, pltpu.roll(ct, t, axis=1), 0.0)
        t *= 2
    prev_cols = ct - running

    sum_ei_logcs = jnp.float32(0.0)
    for g in range(_G):
        sum_ei_logcs = sum_ei_logcs + jnp.sum(
            eis[g] * jnp.log(css[g] + prev_cols))

    wll = sum_ei_lh - n_ev * m - sum_ei_logcs
    loss = jnp.where(n_ev == 0.0, 0.0, -wll / n_ev)
    out_ref[0, 0] = loss


def kernel(log_hazard, event_times, event_indicators):
    lh = log_hazard.reshape(_R, _C)
    et = event_times.reshape(_R, _C)
    ei = event_indicators.reshape(_R, _C)
    out = pl.pallas_call(
        _cox_body,
        out_shape=jax.ShapeDtypeStruct((1, 1), jnp.float32),
        in_specs=[
            pl.BlockSpec(memory_space=pltpu.VMEM),
            pl.BlockSpec(memory_space=pltpu.VMEM),
            pl.BlockSpec(memory_space=pltpu.VMEM),
        ],
        out_specs=pl.BlockSpec(memory_space=pltpu.SMEM),
    )(lh, et, ei)
    return out[0, 0]
